# Initial kernel scaffold; baseline (speedup 1.0000x reference)
import jax
import jax.numpy as jnp
from jax.experimental import pallas as pl


def kernel(user_x, edge_index, W_l0, b_l0, W_r0, W_l1, b_l1, W_r1, gamma0, beta0, gamma1, beta1, ded_table, bil_W, bil_b):
    raise NotImplementedError("write your pallas kernel here")


# speedup vs baseline: 5.2543x; 5.2543x over previous
import functools

import jax
import jax.numpy as jnp
from jax import lax
from jax.experimental import pallas as pl
from jax.experimental.pallas import tpu as pltpu
from jax.experimental.pallas import tpu_sc as plsc

N = 10000
NPAD = 10240
E = 320000
EPAD = 327680
ROWS2D = EPAD // 128
NC = 2
NS = 16
CH = 4
TPW = EPAD // (NC * NS) // (128 * CH)
RPT = NPAD // NS


def _seg_body_deg(z_hbm, src_hbm, dst_hbm, zz_hbm, zd_hbm, ones_hbm,
                  outz_hbm, outd_hbm, srcv, dstv, rowsv, onesv, accz, accd,
                  sem):
    cid = lax.axis_index("c")
    sid = lax.axis_index("s")
    wid = sid * NC + cid
    r0 = sid * RPT
    pltpu.sync_copy(zz_hbm.at[pl.ds(r0, RPT)], accz.at[pl.ds(r0, RPT)])
    pltpu.sync_copy(zd_hbm.at[pl.ds(r0, RPT)], accd.at[pl.ds(r0, RPT)])
    pltpu.sync_copy(ones_hbm, onesv)
    plsc.subcore_barrier()

    base_row = wid * (TPW * CH)

    def step(i, carry):
        rr = base_row + i * CH
        pltpu.sync_copy(src_hbm.at[pl.ds(rr, CH)], srcv)
        pltpu.sync_copy(dst_hbm.at[pl.ds(rr, CH)], dstv)
        cps = [pltpu.async_copy(z_hbm.at[srcv.at[j]], rowsv.at[j], sem)
               for j in range(CH)]
        for cp in cps:
            cp.wait()
        for j in range(CH):
            pltpu.sync_copy(rowsv.at[j], accz.at[dstv.at[j]], add=True)
            pltpu.sync_copy(onesv, accd.at[dstv.at[j]], add=True)
        return carry

    lax.fori_loop(0, TPW, step, 0)
    plsc.subcore_barrier()
    pltpu.sync_copy(accz.at[pl.ds(r0, RPT)], outz_hbm.at[cid, pl.ds(r0, RPT)])
    pltpu.sync_copy(accd.at[pl.ds(r0, RPT)], outd_hbm.at[cid, pl.ds(r0, RPT)])


def _seg_body(z_hbm, src_hbm, dst_hbm, zz_hbm, outz_hbm, srcv, dstv, rowsv,
              accz, sem):
    cid = lax.axis_index("c")
    sid = lax.axis_index("s")
    wid = sid * NC + cid
    r0 = sid * RPT
    pltpu.sync_copy(zz_hbm.at[pl.ds(r0, RPT)], accz.at[pl.ds(r0, RPT)])
    plsc.subcore_barrier()

    base_row = wid * (TPW * CH)

    def step(i, carry):
        rr = base_row + i * CH
        pltpu.sync_copy(src_hbm.at[pl.ds(rr, CH)], srcv)
        pltpu.sync_copy(dst_hbm.at[pl.ds(rr, CH)], dstv)
        cps = [pltpu.async_copy(z_hbm.at[srcv.at[j]], rowsv.at[j], sem)
               for j in range(CH)]
        for cp in cps:
            cp.wait()
        for j in range(CH):
            pltpu.sync_copy(rowsv.at[j], accz.at[dstv.at[j]], add=True)
        return carry

    lax.fori_loop(0, TPW, step, 0)
    plsc.subcore_barrier()
    pltpu.sync_copy(accz.at[pl.ds(r0, RPT)], outz_hbm.at[cid, pl.ds(r0, RPT)])


_SC_MESH = dict(core_axis_name="c", subcore_axis_name="s")

_seg_sum_deg = functools.partial(
    pl.kernel,
    out_type=[jax.ShapeDtypeStruct((NC, NPAD, 64), jnp.float32),
              jax.ShapeDtypeStruct((NC, NPAD, 16), jnp.float32)],
    mesh=plsc.VectorSubcoreMesh(**_SC_MESH),
    compiler_params=pltpu.CompilerParams(use_tc_tiling_on_sc=False),
    scratch_types=[pltpu.VMEM((CH, 128), jnp.int32),
                   pltpu.VMEM((CH, 128), jnp.int32),
                   pltpu.VMEM((CH, 128, 64), jnp.float32),
                   pltpu.VMEM((128, 16), jnp.float32),
                   pltpu.VMEM_SHARED((NPAD, 64), jnp.float32),
                   pltpu.VMEM_SHARED((NPAD, 16), jnp.float32),
                   pltpu.SemaphoreType.DMA],
)(_seg_body_deg)

_seg_sum = functools.partial(
    pl.kernel,
    out_type=[jax.ShapeDtypeStruct((NC, NPAD, 64), jnp.float32)],
    mesh=plsc.VectorSubcoreMesh(**_SC_MESH),
    compiler_params=pltpu.CompilerParams(use_tc_tiling_on_sc=False),
    scratch_types=[pltpu.VMEM((CH, 128), jnp.int32),
                   pltpu.VMEM((CH, 128), jnp.int32),
                   pltpu.VMEM((CH, 128, 64), jnp.float32),
                   pltpu.VMEM_SHARED((NPAD, 64), jnp.float32),
                   pltpu.SemaphoreType.DMA],
)(_seg_body)


def _proj0_body(x_ref, w_ref, z_ref, r_ref):
    y = jnp.dot(x_ref[...], w_ref[...], preferred_element_type=jnp.float32)
    z_ref[:N, :] = y[:, :64]
    z_ref[N:, :] = jnp.zeros((NPAD - N, 64), jnp.float32)
    r_ref[...] = y[:, 64:]


def _mid_body(pz_ref, pd_ref, r_ref, b_ref, g_ref, be_ref, w_ref,
              z_ref, r1_ref, deg_ref):
    agg = pz_ref[0, :N, :] + pz_ref[1, :N, :]
    deg = pd_ref[0, :N, 0:1] + pd_ref[1, :N, 0:1]
    degc = jnp.maximum(deg, 1.0)
    pre = agg / degc + b_ref[...] + r_ref[...]
    mu = jnp.mean(pre, axis=0, keepdims=True)
    var = jnp.mean((pre - mu) ** 2, axis=0, keepdims=True)
    h = (pre - mu) * lax.rsqrt(var + 1e-5) * g_ref[...] + be_ref[...]
    h = jnp.maximum(h, 0.0)
    y = jnp.dot(h, w_ref[...], preferred_element_type=jnp.float32)
    z_ref[:N, :] = y[:, :64]
    z_ref[N:, :] = jnp.zeros((NPAD - N, 64), jnp.float32)
    r1_ref[...] = y[:, 64:]
    deg_ref[...] = degc


def _out_body(pz_ref, deg_ref, r_ref, b_ref, g_ref, be_ref,
              bw_ref, ded_ref, bb_ref, o_ref):
    agg = pz_ref[0, :N, :] + pz_ref[1, :N, :]
    pre = agg / deg_ref[...] + b_ref[...] + r_ref[...]
    mu = jnp.mean(pre, axis=0, keepdims=True)
    var = jnp.mean((pre - mu) ** 2, axis=0, keepdims=True)
    h = (pre - mu) * lax.rsqrt(var + 1e-5) * g_ref[...] + be_ref[...]
    h = jnp.maximum(h, 0.0)
    t = jnp.dot(h, bw_ref[...], preferred_element_type=jnp.float32)
    s = lax.dot_general(t, ded_ref[...], (((1,), (1,)), ((), ())),
                        preferred_element_type=jnp.float32)
    o_ref[...] = jax.nn.sigmoid(s + bb_ref[...])


_proj0 = pl.pallas_call(
    _proj0_body,
    out_shape=[jax.ShapeDtypeStruct((NPAD, 64), jnp.float32),
               jax.ShapeDtypeStruct((N, 64), jnp.float32)],
)

_mid = pl.pallas_call(
    _mid_body,
    out_shape=[jax.ShapeDtypeStruct((NPAD, 64), jnp.float32),
               jax.ShapeDtypeStruct((N, 64), jnp.float32),
               jax.ShapeDtypeStruct((N, 1), jnp.float32)],
)

_outk = pl.pallas_call(
    _out_body,
    out_shape=jax.ShapeDtypeStruct((N, 8), jnp.float32),
)


def kernel(user_x, edge_index, W_l0, b_l0, W_r0, W_l1, b_l1, W_r1,
           gamma0, beta0, gamma1, beta1, ded_table, bil_W, bil_b):
    src = edge_index[0]
    dst = edge_index[1]
    npad_e = EPAD - E
    src_p = jnp.concatenate([src, jnp.zeros((npad_e,), jnp.int32)])
    dst_p = jnp.concatenate([dst, jnp.full((npad_e,), N, jnp.int32)])
    src2d = src_p.reshape(ROWS2D, 128)
    dst2d = dst_p.reshape(ROWS2D, 128)

    zz = jnp.zeros((NPAD, 64), jnp.float32)
    zd = jnp.zeros((NPAD, 16), jnp.float32)
    ones = jnp.ones((128, 16), jnp.float32)

    wcat0 = jnp.concatenate([W_l0, W_r0], axis=1)
    wcat1 = jnp.concatenate([W_l1, W_r1], axis=1)

    z0, r0 = _proj0(user_x, wcat0)
    pz0, pd = _seg_sum_deg(z0, src2d, dst2d, zz, zd, ones)
    z1, r1, degc = _mid(pz0, pd, r0, b_l0.reshape(1, 64),
                        gamma0.reshape(1, 64), beta0.reshape(1, 64), wcat1)
    pz1, = _seg_sum(z1, src2d, dst2d, zz)
    probs = _outk(pz1, degc, r1, b_l1.reshape(1, 64),
                  gamma1.reshape(1, 64), beta1.reshape(1, 64),
                  bil_W, ded_table, bil_b.reshape(1, 1))
    return probs
